# baseline (speedup 1.0000x reference)
import jax
import jax.numpy as jnp
from jax import lax
from jax.experimental import pallas as pl
from jax.experimental.pallas import tpu as pltpu
from jax.experimental.pallas import tpu_sc as plsc

B, N, D = 1024, 1000, 128
NQ = 16
CTX = 2 * D

_info = plsc.get_sparse_core_info()
NC, NS = _info.num_cores, _info.num_subcores
NW = NC * NS
BPW = B // NW
CHUNK = 2
NCHUNKS = BPW // CHUNK
CROWS = CHUNK * NQ
NBUF = 12
LOOKAHEAD = 10


def _tec_body(emb_hbm, fa2_hbm, cn_hbm, w_hbm, out_hbm,
              fa2_v, cn_v, w_v, ifa_v, icn_v, o_v,
              *sems):
    wid = lax.axis_index("s") * NC + lax.axis_index("c")
    b0 = wid * BPW
    gsems = list(sems[:NBUF])
    wsems = list(sems[NBUF:])

    prolog = [
        pltpu.async_copy(fa2_hbm.at[pl.ds(b0, BPW)], fa2_v, gsems[0]),
        pltpu.async_copy(cn_hbm.at[pl.ds(b0, BPW)], cn_v, gsems[1]),
        pltpu.async_copy(w_hbm, w_v, gsems[2]),
    ]
    for h in prolog:
        h.wait()
    phv = [w_v[pl.ds(v * 16, 16)] for v in range(16)]

    def build_idx(c):
        for jj in range(CHUNK):
            j = c * CHUNK + jj
            base = (b0 + j) * N
            ifa_v[c, pl.ds(jj * 16, 16)] = (fa2_v[j] & 0x3FFFFFFF) + base
            icn_v[c, pl.ds(jj * 16, 16)] = cn_v[j] + base

    def fire_gather(c):
        buf = c % NBUF
        return [
            pltpu.async_copy(emb_hbm.at[ifa_v.at[c]],
                             o_v.at[buf, pl.ds(0, CROWS), pl.ds(0, 128)],
                             gsems[buf]),
            pltpu.async_copy(emb_hbm.at[icn_v.at[c]],
                             o_v.at[buf, pl.ds(0, CROWS), pl.ds(128, 128)],
                             gsems[buf]),
        ]

    gh = {}
    for c in range(LOOKAHEAD):
        build_idx(c)
        gh[c] = fire_gather(c)
    for c in range(LOOKAHEAD, NCHUNKS):
        build_idx(c)
    wh = {}
    for c in range(NCHUNKS):
        buf = c % NBUF
        if c + LOOKAHEAD < NCHUNKS:
            if c - 1 in wh:
                for h in wh.pop(c - 1):
                    h.wait()
            gh[c + LOOKAHEAD] = fire_gather(c + LOOKAHEAD)
        for h in gh.pop(c):
            h.wait()

        @plsc.parallel_loop(0, CROWS, unroll=2)
        def _blend(t):
            jb = c * CHUNK + t // NQ
            swb = fa2_v[jb] >= (1 << 30)
            for v in range(16):
                x = o_v[buf, t, pl.ds(v * 16, 16)]
                o_v[buf, t, pl.ds(v * 16, 16)] = jnp.where(swb, phv[v], x)

        wh[c] = [pltpu.async_copy(
            o_v.at[buf],
            out_hbm.at[pl.ds((b0 + c * CHUNK) * NQ, CROWS)],
            wsems[buf])]
    for c in list(wh):
        for h in wh.pop(c):
            h.wait()


@jax.jit
def _tsp_context_sc(emb_flat, fa2, cn, w):
    mesh = plsc.VectorSubcoreMesh(core_axis_name="c", subcore_axis_name="s")
    run = pl.kernel(
        _tec_body,
        mesh=mesh,
        out_type=jax.ShapeDtypeStruct((B * NQ, CTX), jnp.float32),
        scratch_types=[
            pltpu.VMEM((BPW, NQ), jnp.int32),
            pltpu.VMEM((BPW, NQ), jnp.int32),
            pltpu.VMEM((256,), jnp.float32),
            pltpu.VMEM((NCHUNKS, CROWS), jnp.int32),
            pltpu.VMEM((NCHUNKS, CROWS), jnp.int32),
            pltpu.VMEM((NBUF, CROWS, CTX), jnp.float32),
        ] + [pltpu.SemaphoreType.DMA] * (2 * NBUF),
    )
    return run(emb_flat, fa2, cn, w)


def kernel(embeddings, first_a, current_node, is_initial_action, W_placeholder):
    emb_flat = embeddings.reshape(B * N, D)
    fa2 = first_a.astype(jnp.int32) | (
        is_initial_action.astype(jnp.int32)[:, None] << 30)
    out = _tsp_context_sc(emb_flat, fa2, current_node.astype(jnp.int32),
                          W_placeholder)
    return out.reshape(B, NQ, CTX)
